# baseline (speedup 1.0000x reference)
import functools

import jax
import jax.numpy as jnp
from jax import lax
from jax.experimental import pallas as pl
from jax.experimental.pallas import tpu as pltpu
from jax.experimental.pallas import tpu_sc as plsc

N = 32768
E = 8192
D = 256

BLK_N = 512
N_STEPS = N // BLK_N
LANES = 128
MM_CHUNKS = 8


def _argmin_body(x_ref, ebt_ref, xp_ref, ep_ref, code_ref):
    xb = x_ref[...].astype(jnp.bfloat16)
    xp = xp_ref[...]

    val = jnp.full((BLK_N, LANES), jnp.inf, jnp.float32)
    grp = jnp.zeros((BLK_N, LANES), jnp.float32)
    CB = E // MM_CHUNKS
    for c in range(MM_CHUNKS):
        mmn = jax.lax.dot_general(
            xb, ebt_ref[:, c * CB:(c + 1) * CB], (((1,), (0,)), ((), ())),
            preferred_element_type=jnp.float32)
        for gc in range(CB // LANES):
            g = c * (CB // LANES) + gc
            sl = slice(gc * LANES, (gc + 1) * LANES)
            esl = slice(g * LANES, (g + 1) * LANES)
            d_g = mmn[:, sl] + (xp + ep_ref[:, esl])
            better = d_g < val
            val = jnp.minimum(val, d_g)
            grp = jnp.where(better, jnp.float32(g), grp)

    lane = jax.lax.broadcasted_iota(jnp.int32, (BLK_N, LANES), 1).astype(jnp.float32)
    gidx = grp * jnp.float32(LANES) + lane
    bm = jnp.min(val, axis=1, keepdims=True)
    bidx = jnp.min(jnp.where(val == bm, gidx, jnp.inf), axis=1)
    code_ref[...] = bidx.astype(jnp.int32)


def _argmin_call(x, ebt, xp, ep):
    n = x.shape[0]
    return pl.pallas_call(
        _argmin_body,
        grid=(n // BLK_N,),
        in_specs=[
            pl.BlockSpec((BLK_N, D), lambda i: (i, 0)),
            pl.BlockSpec((D, E), lambda i: (0, 0)),
            pl.BlockSpec((BLK_N, 1), lambda i: (i, 0)),
            pl.BlockSpec((1, E), lambda i: (0, 0)),
        ],
        out_specs=pl.BlockSpec((BLK_N,), lambda i: (i,)),
        out_shape=jax.ShapeDtypeStruct((n,), jnp.int32),
    )(x, ebt, xp, ep)


def _make_gather(n):
    info = plsc.get_sparse_core_info()
    NC, NS = info.num_cores, info.num_subcores
    NW = NC * NS
    b_per_w = n // NW
    CHUNK = 256
    n_chunks = b_per_w // CHUNK

    mesh = plsc.VectorSubcoreMesh(core_axis_name="c", subcore_axis_name="s")

    @functools.partial(
        pl.kernel, mesh=mesh,
        out_type=jax.ShapeDtypeStruct((n, D), jnp.float32),
        scratch_types=[
            pltpu.VMEM((CHUNK,), jnp.int32),
            pltpu.VMEM((CHUNK, D), jnp.float32),
            pltpu.SemaphoreType.DMA,
        ],
    )
    def gather(table_hbm, idx_hbm, out_hbm, idx_v, rows_v, sem):
        wid = lax.axis_index("s") * NC + lax.axis_index("c")
        base = wid * b_per_w

        def chunk(c, _):
            off = base + c * CHUNK
            pltpu.sync_copy(idx_hbm.at[pl.ds(off, CHUNK)], idx_v)
            pltpu.async_copy(table_hbm.at[idx_v], rows_v, sem).wait()
            pltpu.sync_copy(rows_v, out_hbm.at[pl.ds(off, CHUNK)])
            return 0

        jax.lax.fori_loop(0, n_chunks, chunk, 0)

    return gather


def kernel(inputs, embeddings):
    x = inputs
    e = embeddings
    et = e.T
    xp = jnp.sum(jnp.power(x, 2), axis=-1, keepdims=True)
    ep = jnp.sum(jnp.power(et, 2), axis=0, keepdims=True)
    ebt = (et * jnp.float32(-2.0)).astype(jnp.bfloat16)
    h = N // 2
    gather = _make_gather(h)
    code1 = _argmin_call(x[:h], ebt, xp[:h], ep)
    quant1 = gather(e, code1)
    code2 = _argmin_call(x[h:], ebt, xp[h:], ep)
    quant2 = gather(e, code2)
    code = jnp.concatenate([code1, code2])
    quant = jnp.concatenate([quant1, quant2])
    return (code, quant)


# speedup vs baseline: 1.1007x; 1.1007x over previous
import functools

import jax
import jax.numpy as jnp
from jax import lax
from jax.experimental import pallas as pl
from jax.experimental.pallas import tpu as pltpu
from jax.experimental.pallas import tpu_sc as plsc

N = 32768
E = 8192
D = 256

BLK_N = 512
N_STEPS = N // BLK_N
LANES = 128
MM_CHUNKS = 8


def _argmin_body(x_ref, ebt_ref, xp_ref, ep_ref, code_ref):
    xb = x_ref[...].astype(jnp.bfloat16)
    xp = xp_ref[...]

    val = jnp.full((BLK_N, LANES), jnp.inf, jnp.float32)
    grp = jnp.zeros((BLK_N, LANES), jnp.float32)
    CB = E // MM_CHUNKS
    for c in range(MM_CHUNKS):
        mmn = jax.lax.dot_general(
            xb, ebt_ref[:, c * CB:(c + 1) * CB], (((1,), (0,)), ((), ())),
            preferred_element_type=jnp.float32)
        for gc in range(CB // LANES):
            g = c * (CB // LANES) + gc
            sl = slice(gc * LANES, (gc + 1) * LANES)
            esl = slice(g * LANES, (g + 1) * LANES)
            d_g = mmn[:, sl] + (xp + ep_ref[:, esl])
            better = d_g < val
            val = jnp.minimum(val, d_g)
            grp = jnp.where(better, jnp.float32(g), grp)

    lane = jax.lax.broadcasted_iota(jnp.int32, (BLK_N, LANES), 1).astype(jnp.float32)
    gidx = grp * jnp.float32(LANES) + lane
    bm = jnp.min(val, axis=1, keepdims=True)
    bidx = jnp.min(jnp.where(val == bm, gidx, jnp.inf), axis=1)
    code_ref[...] = bidx.astype(jnp.int32)


def _argmin_call(x, ebt, xp, ep):
    n = x.shape[0]
    return pl.pallas_call(
        _argmin_body,
        grid=(n // BLK_N,),
        in_specs=[
            pl.BlockSpec((BLK_N, D), lambda i: (i, 0)),
            pl.BlockSpec((D, E), lambda i: (0, 0)),
            pl.BlockSpec((BLK_N, 1), lambda i: (i, 0)),
            pl.BlockSpec((1, E), lambda i: (0, 0)),
        ],
        out_specs=pl.BlockSpec((BLK_N,), lambda i: (i,)),
        out_shape=jax.ShapeDtypeStruct((n,), jnp.int32),
    )(x, ebt, xp, ep)


def _make_gather(n):
    info = plsc.get_sparse_core_info()
    NC, NS = info.num_cores, info.num_subcores
    NW = NC * NS
    b_per_w = n // NW
    CHUNK = 128
    n_chunks = b_per_w // CHUNK

    mesh = plsc.VectorSubcoreMesh(core_axis_name="c", subcore_axis_name="s")

    @functools.partial(
        pl.kernel, mesh=mesh,
        out_type=jax.ShapeDtypeStruct((n, D), jnp.float32),
        scratch_types=[
            pltpu.VMEM((2, CHUNK), jnp.int32),
            pltpu.VMEM((CHUNK, D), jnp.float32),
            pltpu.VMEM((CHUNK, D), jnp.float32),
            pltpu.SemaphoreType.DMA,
            pltpu.SemaphoreType.DMA,
            pltpu.SemaphoreType.DMA,
            pltpu.SemaphoreType.DMA,
        ],
    )
    def gather(table_hbm, idx_hbm, out_hbm, idx_v, rows_a, rows_b,
               gs_a, gs_b, ss_a, ss_b):
        wid = lax.axis_index("s") * NC + lax.axis_index("c")
        base = wid * b_per_w
        rows = (rows_a, rows_b)
        gsem = (gs_a, gs_b)
        ssem = (ss_a, ss_b)

        def fire(c):
            b = c % 2
            off = base + c * CHUNK
            pltpu.sync_copy(idx_hbm.at[pl.ds(off, CHUNK)], idx_v.at[b])
            return pltpu.async_copy(table_hbm.at[idx_v.at[b]], rows[b], gsem[b])

        g = [fire(0), fire(1)]
        st = [None, None]
        for c in range(n_chunks):
            b = c % 2
            g[b].wait()
            st[b] = pltpu.async_copy(
                rows[b], out_hbm.at[pl.ds(base + c * CHUNK, CHUNK)], ssem[b])
            if c + 2 < n_chunks:
                st[b].wait()
                g[b] = fire(c + 2)
        for b in range(2):
            if st[b] is not None:
                st[b].wait()

    return gather


def kernel(inputs, embeddings):
    x = inputs
    e = embeddings
    et = e.T
    xp = jnp.sum(jnp.power(x, 2), axis=-1, keepdims=True)
    ep = jnp.sum(jnp.power(et, 2), axis=0, keepdims=True)
    ebt = (et * jnp.float32(-2.0)).astype(jnp.bfloat16)
    code = _argmin_call(x, ebt, xp, ep)
    quant = _make_gather(N)(e, code)
    return (code, quant)


# speedup vs baseline: 1.1011x; 1.0004x over previous
import functools

import jax
import jax.numpy as jnp
from jax import lax
from jax.experimental import pallas as pl
from jax.experimental.pallas import tpu as pltpu
from jax.experimental.pallas import tpu_sc as plsc

N = 32768
E = 8192
D = 256

BLK_N = 512
N_STEPS = N // BLK_N
LANES = 128
MM_CHUNKS = 8


def _argmin_body(x_ref, ebt_ref, xp_ref, ep_ref, code_ref):
    xb = x_ref[...].astype(jnp.bfloat16)
    xp = xp_ref[...]

    val = jnp.full((BLK_N, LANES), jnp.inf, jnp.float32)
    grp = jnp.zeros((BLK_N, LANES), jnp.float32)
    CB = E // MM_CHUNKS
    for c in range(MM_CHUNKS):
        mmn = jax.lax.dot_general(
            xb, ebt_ref[:, c * CB:(c + 1) * CB], (((1,), (0,)), ((), ())),
            preferred_element_type=jnp.float32)
        for gc in range(CB // LANES):
            g = c * (CB // LANES) + gc
            sl = slice(gc * LANES, (gc + 1) * LANES)
            esl = slice(g * LANES, (g + 1) * LANES)
            d_g = mmn[:, sl] + (xp + ep_ref[:, esl])
            better = d_g < val
            val = jnp.minimum(val, d_g)
            grp = jnp.where(better, jnp.float32(g), grp)

    lane = jax.lax.broadcasted_iota(jnp.int32, (BLK_N, LANES), 1).astype(jnp.float32)
    gidx = grp * jnp.float32(LANES) + lane
    bm = jnp.min(val, axis=1, keepdims=True)
    bidx = jnp.min(jnp.where(val == bm, gidx, jnp.inf), axis=1)
    code_ref[...] = bidx.astype(jnp.int32)


def _argmin_call(x, ebt, xp, ep):
    n = x.shape[0]
    return pl.pallas_call(
        _argmin_body,
        grid=(n // BLK_N,),
        in_specs=[
            pl.BlockSpec((BLK_N, D), lambda i: (i, 0)),
            pl.BlockSpec(memory_space=pltpu.VMEM),
            pl.BlockSpec((BLK_N, 1), lambda i: (i, 0)),
            pl.BlockSpec(memory_space=pltpu.VMEM),
        ],
        out_specs=pl.BlockSpec((BLK_N,), lambda i: (i,)),
        out_shape=jax.ShapeDtypeStruct((n,), jnp.int32),
    )(x, ebt, xp, ep)


def _make_gather(n):
    info = plsc.get_sparse_core_info()
    NC, NS = info.num_cores, info.num_subcores
    NW = NC * NS
    b_per_w = n // NW
    CHUNK = 128
    n_chunks = b_per_w // CHUNK

    mesh = plsc.VectorSubcoreMesh(core_axis_name="c", subcore_axis_name="s")

    @functools.partial(
        pl.kernel, mesh=mesh,
        out_type=jax.ShapeDtypeStruct((n, D), jnp.float32),
        scratch_types=[
            pltpu.VMEM((2, CHUNK), jnp.int32),
            pltpu.VMEM((CHUNK, D), jnp.float32),
            pltpu.VMEM((CHUNK, D), jnp.float32),
            pltpu.SemaphoreType.DMA,
            pltpu.SemaphoreType.DMA,
            pltpu.SemaphoreType.DMA,
            pltpu.SemaphoreType.DMA,
        ],
    )
    def gather(table_hbm, idx_hbm, out_hbm, idx_v, rows_a, rows_b,
               gs_a, gs_b, ss_a, ss_b):
        wid = lax.axis_index("s") * NC + lax.axis_index("c")
        base = wid * b_per_w
        rows = (rows_a, rows_b)
        gsem = (gs_a, gs_b)
        ssem = (ss_a, ss_b)

        def fire(c):
            b = c % 2
            off = base + c * CHUNK
            pltpu.sync_copy(idx_hbm.at[pl.ds(off, CHUNK)], idx_v.at[b])
            return pltpu.async_copy(table_hbm.at[idx_v.at[b]], rows[b], gsem[b])

        g = [fire(0), fire(1)]
        st = [None, None]
        for c in range(n_chunks):
            b = c % 2
            g[b].wait()
            st[b] = pltpu.async_copy(
                rows[b], out_hbm.at[pl.ds(base + c * CHUNK, CHUNK)], ssem[b])
            if c + 2 < n_chunks:
                st[b].wait()
                g[b] = fire(c + 2)
        for b in range(2):
            if st[b] is not None:
                st[b].wait()

    return gather


def kernel(inputs, embeddings):
    x = inputs
    e = embeddings
    et = e.T
    xp = jnp.sum(jnp.power(x, 2), axis=-1, keepdims=True)
    ep = jnp.sum(jnp.power(et, 2), axis=0, keepdims=True)
    ebt = (et * jnp.float32(-2.0)).astype(jnp.bfloat16)
    code = _argmin_call(x, ebt, xp, ep)
    quant = _make_gather(N)(e, code)
    return (code, quant)


# speedup vs baseline: 1.1786x; 1.0704x over previous
import functools

import jax
import jax.numpy as jnp
from jax import lax
from jax.experimental import pallas as pl
from jax.experimental.pallas import tpu as pltpu
from jax.experimental.pallas import tpu_sc as plsc

N = 32768
E = 8192
D = 256

BLK_N = 512
N_STEPS = N // BLK_N
LANES = 128
MM_CHUNKS = 8


def _argmin_body(x_ref, ebt_ref, xp_ref, ep_ref, code_ref):
    xb = x_ref[...].astype(jnp.bfloat16)
    xp = xp_ref[...]

    val = jnp.full((BLK_N, LANES), jnp.inf, jnp.float32)
    grp = jnp.zeros((BLK_N, LANES), jnp.float32)
    CB = E // MM_CHUNKS
    for c in range(MM_CHUNKS):
        mmn = jax.lax.dot_general(
            xb, ebt_ref[c * CB:(c + 1) * CB, :], (((1,), (1,)), ((), ())),
            preferred_element_type=jnp.float32)
        for gc in range(CB // LANES):
            g = c * (CB // LANES) + gc
            sl = slice(gc * LANES, (gc + 1) * LANES)
            esl = slice(g * LANES, (g + 1) * LANES)
            d_g = mmn[:, sl] + (xp + ep_ref[:, esl])
            better = d_g < val
            val = jnp.minimum(val, d_g)
            grp = jnp.where(better, jnp.float32(g), grp)

    lane = jax.lax.broadcasted_iota(jnp.int32, (BLK_N, LANES), 1).astype(jnp.float32)
    gidx = grp * jnp.float32(LANES) + lane
    bm = jnp.min(val, axis=1, keepdims=True)
    bidx = jnp.min(jnp.where(val == bm, gidx, jnp.inf), axis=1)
    code_ref[...] = bidx.astype(jnp.int32)


def _argmin_call(x, ebt, xp, ep):
    n = x.shape[0]
    return pl.pallas_call(
        _argmin_body,
        grid=(n // BLK_N,),
        in_specs=[
            pl.BlockSpec((BLK_N, D), lambda i: (i, 0)),
            pl.BlockSpec(memory_space=pltpu.VMEM),
            pl.BlockSpec((BLK_N, 1), lambda i: (i, 0)),
            pl.BlockSpec(memory_space=pltpu.VMEM),
        ],
        out_specs=pl.BlockSpec((BLK_N,), lambda i: (i,)),
        out_shape=jax.ShapeDtypeStruct((n,), jnp.int32),
    )(x, ebt, xp, ep)


def _make_gather(n):
    info = plsc.get_sparse_core_info()
    NC, NS = info.num_cores, info.num_subcores
    NW = NC * NS
    b_per_w = n // NW
    CHUNK = 128
    n_chunks = b_per_w // CHUNK

    mesh = plsc.VectorSubcoreMesh(core_axis_name="c", subcore_axis_name="s")

    @functools.partial(
        pl.kernel, mesh=mesh,
        out_type=jax.ShapeDtypeStruct((n, D), jnp.float32),
        scratch_types=[
            pltpu.VMEM((2, CHUNK), jnp.int32),
            pltpu.VMEM((CHUNK, D), jnp.float32),
            pltpu.VMEM((CHUNK, D), jnp.float32),
            pltpu.SemaphoreType.DMA,
            pltpu.SemaphoreType.DMA,
            pltpu.SemaphoreType.DMA,
            pltpu.SemaphoreType.DMA,
        ],
    )
    def gather(table_hbm, idx_hbm, out_hbm, idx_v, rows_a, rows_b,
               gs_a, gs_b, ss_a, ss_b):
        wid = lax.axis_index("s") * NC + lax.axis_index("c")
        base = wid * b_per_w
        rows = (rows_a, rows_b)
        gsem = (gs_a, gs_b)
        ssem = (ss_a, ss_b)

        def fire(c):
            b = c % 2
            off = base + c * CHUNK
            pltpu.sync_copy(idx_hbm.at[pl.ds(off, CHUNK)], idx_v.at[b])
            return pltpu.async_copy(table_hbm.at[idx_v.at[b]], rows[b], gsem[b])

        g = [fire(0), fire(1)]
        st = [None, None]
        for c in range(n_chunks):
            b = c % 2
            g[b].wait()
            st[b] = pltpu.async_copy(
                rows[b], out_hbm.at[pl.ds(base + c * CHUNK, CHUNK)], ssem[b])
            if c + 2 < n_chunks:
                st[b].wait()
                g[b] = fire(c + 2)
        for b in range(2):
            if st[b] is not None:
                st[b].wait()

    return gather


def kernel(inputs, embeddings):
    x = inputs
    e = embeddings
    et = e.T
    xp = jnp.sum(jnp.power(x, 2), axis=-1, keepdims=True)
    ep = jnp.sum(jnp.power(et, 2), axis=0, keepdims=True)
    ebn = (e * jnp.float32(-2.0)).astype(jnp.bfloat16)
    code = _argmin_call(x, ebn, xp, ep)
    quant = _make_gather(N)(e, code)
    return (code, quant)
